# baseline (speedup 1.0000x reference)
import functools

import jax
import jax.numpy as jnp
from jax import lax
from jax.experimental import pallas as pl
from jax.experimental.pallas import tpu as pltpu
from jax.experimental.pallas import tpu_sc as plsc

_CHANNELS = 16
_SCALAR_MAX = 4
_BASIS = 9
_OUT_W = 2 * _BASIS * _CHANNELS
_NPAIR = 81

_AN_IDX = jnp.array([0, 0, 0, 0, 0, 0, 1, 2, 3], jnp.int32)
_AN_VALID = jnp.array([False, True, False, False, False, False, True, True, True])
_SDIMS = jnp.array([3, 4, 4, 4], jnp.int32)

_C = 128
_LANES = 16


def _build_table(w):
    ia = _AN_IDX[:, None]
    ib = _AN_IDX[None, :]
    valid = _AN_VALID[:, None] & _AN_VALID[None, :]
    sfa = w[ia * 4 + ib].reshape(9, 9, _SCALAR_MAX, _CHANNELS)
    sfb = w[ib * 4 + ia].reshape(9, 9, _SCALAR_MAX, _CHANNELS)
    pad = ((0, 0), (0, 0), (0, _BASIS - _SCALAR_MAX), (0, 0))
    sfa_p = jnp.pad(sfa, pad)
    sfb_p = jnp.pad(sfb, pad)
    rows = jnp.arange(_BASIS)[None, None, :, None]
    ma = valid[:, :, None, None] & (rows < _SDIMS[ia][:, :, None, None])
    mb = valid[:, :, None, None] & (rows < _SDIMS[ib][:, :, None, None])
    ta = jnp.where(ma, sfa_p, 0.0)
    tb = jnp.where(mb, sfb_p, 0.0)
    return jnp.concatenate([ta, tb], axis=-1).reshape(_NPAIR, _OUT_W)


def _sc_kernel(num_workers, n_atoms, e_total):
    epw = e_total // num_workers
    nchunks = pl.cdiv(epw, _C)
    nvec = epw // _LANES
    mesh = plsc.VectorSubcoreMesh(core_axis_name="c", subcore_axis_name="s")

    @functools.partial(
        pl.kernel,
        mesh=mesh,
        compiler_params=pltpu.CompilerParams(use_tc_tiling_on_sc=False),
        out_type=jax.ShapeDtypeStruct((e_total, _OUT_W), jnp.float32),
        scratch_types=[
            pltpu.VMEM((epw,), jnp.int32),
            pltpu.VMEM((epw,), jnp.int32),
            pltpu.VMEM((epw,), jnp.int32),
            pltpu.VMEM((epw,), jnp.int32),
            pltpu.VMEM((epw,), jnp.int32),
            pltpu.VMEM((_C, _OUT_W), jnp.float32),
            pltpu.VMEM((_C, _OUT_W), jnp.float32),
            pltpu.SemaphoreType.DMA,
            pltpu.SemaphoreType.DMA,
            pltpu.SemaphoreType.DMA,
            pltpu.SemaphoreType.DMA,
        ],
    )
    def body(an_hbm, eidx_hbm, table_hbm, out_hbm,
             i0_v, i1_v, a0_v, a1_v, pair_v, rows0_v, rows1_v,
             sg0, sg1, so0, so1):
        wid = lax.axis_index("s") * 2 + lax.axis_index("c")
        ebase = wid * epw
        pltpu.sync_copy(eidx_hbm.at[0, pl.ds(ebase, epw)], i0_v)
        pltpu.sync_copy(eidx_hbm.at[1, pl.ds(ebase, epw)], i1_v)

        an_offs = [min(k * _C, epw - _C) for k in range(nchunks)]
        an_descs = []
        for off in an_offs:
            sl = pl.ds(off, _C)
            an_descs.append(pltpu.async_copy(
                an_hbm.at[i0_v.at[sl]], a0_v.at[sl], sg0))
            an_descs.append(pltpu.async_copy(
                an_hbm.at[i1_v.at[sl]], a1_v.at[sl], sg1))
        for d in an_descs:
            d.wait()

        def pair_body(i, carry):
            sl = pl.ds(i * _LANES, _LANES)
            pair_v[sl] = a0_v[sl] * 9 + a1_v[sl]
            return carry

        lax.fori_loop(0, nvec, pair_body, 0)
        if epw % _LANES:
            sl = pl.ds(epw - _LANES, _LANES)
            pair_v[sl] = a0_v[sl] * 9 + a1_v[sl]

        rows = (rows0_v, rows1_v)
        sg = (sg0, sg1)
        so = (so0, so1)
        offs = [min(c * _C, epw - _C) for c in range(nchunks)]

        gd, od = {}, {}

        def out_start(c):
            b = c & 1
            gd[c].wait()
            od[c] = pltpu.async_copy(
                rows[b], out_hbm.at[pl.ds(ebase + offs[c], _C)], so[b])

        for c in range(nchunks):
            b = c & 1
            if c >= 2:
                od[c - 2].wait()
            gd[c] = pltpu.async_copy(
                table_hbm.at[pair_v.at[pl.ds(offs[c], _C)]], rows[b], sg[b])
            if c >= 1:
                out_start(c - 1)
        out_start(nchunks - 1)
        od[nchunks - 2].wait()
        od[nchunks - 1].wait()

    return body


def kernel(atomic_numbers, edge_index, embedding_weight):
    n_atoms = atomic_numbers.shape[0]
    e_total = edge_index.shape[1]
    table = _build_table(embedding_weight)
    info = plsc.get_sparse_core_info()
    num_workers = info.num_cores * info.num_subcores
    if e_total % (num_workers * 8) or e_total // num_workers < _C:
        raise ValueError("unsupported edge count")
    out = _sc_kernel(num_workers, n_atoms, e_total)(
        atomic_numbers, edge_index, table)
    return (out.reshape(e_total, _BASIS, 2 * _CHANNELS), edge_index)
